# baseline (speedup 1.0000x reference)
import functools

import jax
import jax.numpy as jnp
from jax import lax
from jax.experimental import pallas as pl
from jax.experimental.pallas import tpu as pltpu
from jax.experimental.pallas import tpu_sc as plsc

_NC = 2
_NS = 16
_NW = _NC * _NS

_CH = 128
_NBUF = 6
_SKEW = 3
_BLK = 8


@functools.lru_cache(maxsize=None)
def _make_gather(V, D, N):
    assert N % (_NW * _CH * _BLK) == 0
    rows_per_w = N // _NW
    n_chunks = rows_per_w // _CH
    n_blocks = n_chunks // _BLK

    mesh = plsc.VectorSubcoreMesh(core_axis_name="c", subcore_axis_name="s")

    @functools.partial(
        pl.kernel,
        mesh=mesh,
        out_type=jax.ShapeDtypeStruct((N, D), jnp.float32),
        scratch_types=[
            pltpu.VMEM((3, _BLK, _CH), jnp.int32),
            pltpu.VMEM((_NBUF, _CH, D), jnp.float32),
            pltpu.SemaphoreType.DMA((3,)),
            pltpu.SemaphoreType.DMA((_NBUF,)),
            pltpu.SemaphoreType.DMA((_NBUF,)),
        ],
    )
    def k(table_hbm, idx_hbm, out_hbm, idx_v, rows_v, isem, gsem, wsem):
        wid = lax.axis_index("s") * _NC + lax.axis_index("c")
        row0 = wid * rows_per_w
        iblk0 = wid * n_chunks

        def idx_load(blk, buf):
            return pltpu.make_async_copy(
                idx_hbm.at[pl.ds(iblk0 + blk * _BLK, _BLK)],
                idx_v.at[buf],
                isem.at[buf],
            )

        def gather(blk_buf, j, b):
            return pltpu.make_async_copy(
                table_hbm.at[idx_v.at[blk_buf, j]], rows_v.at[b], gsem.at[b]
            )

        def write(g, b):
            return pltpu.make_async_copy(
                rows_v.at[b], out_hbm.at[pl.ds(row0 + g * _CH, _CH)], wsem.at[b]
            )

        idx_load(0, 0).start()
        idx_load(1, 1).start()
        idx_load(2, 2).start()
        idx_load(0, 0).wait()
        for j in range(_BLK):
            b = j % _NBUF
            if j >= _NBUF:
                write(j - _NBUF, b).wait()
            gather(0, j, b).start()
            if j >= _SKEW:
                bw = (j - _SKEW) % _NBUF
                gather(0, j - _SKEW, bw).wait()
                write(j - _SKEW, bw).start()

        def block_body(t, carry):
            ib = t % 3
            idx_load(t, ib).wait()
            for j in range(_BLK):
                g = t * _BLK + j
                b = lax.rem(g, _NBUF)
                write(g - _NBUF, b).wait()
                gather(ib, j, b).start()
                bw = lax.rem(g - _SKEW, _NBUF)
                gather(ib, j, bw).wait()
                write(g - _SKEW, bw).start()
                if j == 3:
                    @pl.when(t < n_blocks - 2)
                    def _():
                        idx_load(t + 2, (t + 2) % 3).start()
            return carry

        lax.fori_loop(1, n_blocks, block_body, 0)

        for c in range(n_chunks - _SKEW, n_chunks):
            b = c % _NBUF
            gather(0, 0, b).wait()
            write(c, b).start()
        for c in range(n_chunks - _NBUF, n_chunks):
            write(c, c % _NBUF).wait()

    return k


def kernel(indices, table):
    Bq, S = indices.shape
    V, D = table.shape
    N = Bq * S
    idx2d = indices.reshape(N // _CH, _CH).astype(jnp.int32)
    out = _make_gather(V, D, N)(table.astype(jnp.float32), idx2d)
    return out.reshape(Bq, S, D)


# speedup vs baseline: 2.7869x; 2.7869x over previous
import functools

import jax
import jax.numpy as jnp
from jax import lax
from jax.experimental import pallas as pl
from jax.experimental.pallas import tpu as pltpu
from jax.experimental.pallas import tpu_sc as plsc

_NC = 2
_NS = 16
_NW = _NC * _NS

_CH = 128
_NBUF = 6
_SKEW = 3
_BLK = 8


@functools.lru_cache(maxsize=None)
def _make_gather(V, D, N):
    assert N % (_NW * _CH * _BLK) == 0
    rows_per_w = N // _NW
    n_chunks = rows_per_w // _CH
    n_blocks = n_chunks // _BLK

    mesh = plsc.VectorSubcoreMesh(core_axis_name="c", subcore_axis_name="s")

    @functools.partial(
        pl.kernel,
        mesh=mesh,
        out_type=jax.ShapeDtypeStruct((N, D), jnp.float32),
        scratch_types=[
            pltpu.VMEM((3, _BLK, _CH), jnp.int32),
            pltpu.VMEM((_NBUF, _CH, D), jnp.float32),
            pltpu.VMEM_SHARED((V, D), jnp.float32),
            pltpu.SemaphoreType.DMA((3,)),
            pltpu.SemaphoreType.DMA((_NBUF,)),
            pltpu.SemaphoreType.DMA((_NBUF,)),
        ],
    )
    def k(table_hbm, idx_hbm, out_hbm, idx_v, rows_v, tab_sh, isem, gsem, wsem):
        wid = lax.axis_index("s") * _NC + lax.axis_index("c")
        row0 = wid * rows_per_w
        iblk0 = wid * n_chunks

        def idx_load(blk, buf):
            return pltpu.make_async_copy(
                idx_hbm.at[pl.ds(iblk0 + blk * _BLK, _BLK)],
                idx_v.at[buf],
                isem.at[buf],
            )

        def gather(blk_buf, j, b):
            return pltpu.make_async_copy(
                tab_sh.at[idx_v.at[blk_buf, j]], rows_v.at[b], gsem.at[b]
            )

        def write(g, b):
            return pltpu.make_async_copy(
                rows_v.at[b], out_hbm.at[pl.ds(row0 + g * _CH, _CH)], wsem.at[b]
            )

        @pl.when(lax.axis_index("s") == 0)
        def _():
            pltpu.sync_copy(table_hbm, tab_sh)
        plsc.subcore_barrier()

        idx_load(0, 0).start()
        idx_load(1, 1).start()
        idx_load(2, 2).start()
        idx_load(0, 0).wait()
        for j in range(_BLK):
            b = j % _NBUF
            if j >= _NBUF:
                write(j - _NBUF, b).wait()
            gather(0, j, b).start()
            if j >= _SKEW:
                bw = (j - _SKEW) % _NBUF
                gather(0, j - _SKEW, bw).wait()
                write(j - _SKEW, bw).start()

        def block_body(t, carry):
            ib = t % 3
            idx_load(t, ib).wait()
            for j in range(_BLK):
                g = t * _BLK + j
                b = lax.rem(g, _NBUF)
                write(g - _NBUF, b).wait()
                gather(ib, j, b).start()
                bw = lax.rem(g - _SKEW, _NBUF)
                gather(ib, j, bw).wait()
                write(g - _SKEW, bw).start()
                if j == 3:
                    @pl.when(t < n_blocks - 2)
                    def _():
                        idx_load(t + 2, (t + 2) % 3).start()
            return carry

        lax.fori_loop(1, n_blocks, block_body, 0)

        for c in range(n_chunks - _SKEW, n_chunks):
            b = c % _NBUF
            gather(0, 0, b).wait()
            write(c, b).start()
        for c in range(n_chunks - _NBUF, n_chunks):
            write(c, c % _NBUF).wait()

    return k


def kernel(indices, table):
    Bq, S = indices.shape
    V, D = table.shape
    N = Bq * S
    idx2d = indices.reshape(N // _CH, _CH).astype(jnp.int32)
    out = _make_gather(V, D, N)(table.astype(jnp.float32), idx2d)
    return out.reshape(Bq, S, D)


# speedup vs baseline: 2.7950x; 1.0029x over previous
import functools

import jax
import jax.numpy as jnp
from jax import lax
from jax.experimental import pallas as pl
from jax.experimental.pallas import tpu as pltpu
from jax.experimental.pallas import tpu_sc as plsc

_NC = 2
_NS = 16
_NW = _NC * _NS

_CH = 128
_NBUF = 7
_SKEW = 3
_BLK = 8


@functools.lru_cache(maxsize=None)
def _make_gather(V, D, N):
    assert N % (_NW * _CH * _BLK) == 0
    rows_per_w = N // _NW
    n_chunks = rows_per_w // _CH
    n_blocks = n_chunks // _BLK

    mesh = plsc.VectorSubcoreMesh(core_axis_name="c", subcore_axis_name="s")

    @functools.partial(
        pl.kernel,
        mesh=mesh,
        out_type=jax.ShapeDtypeStruct((N, D), jnp.float32),
        scratch_types=[
            pltpu.VMEM((3, _BLK, _CH), jnp.int32),
            pltpu.VMEM((_NBUF, _CH, D), jnp.float32),
            pltpu.VMEM_SHARED((V, D), jnp.float32),
            pltpu.SemaphoreType.DMA((3,)),
            pltpu.SemaphoreType.DMA((_NBUF,)),
            pltpu.SemaphoreType.DMA((_NBUF,)),
        ],
    )
    def k(table_hbm, idx_hbm, out_hbm, idx_v, rows_v, tab_sh, isem, gsem, wsem):
        wid = lax.axis_index("s") * _NC + lax.axis_index("c")
        row0 = wid * rows_per_w
        iblk0 = wid * n_chunks

        def idx_load(blk, buf):
            return pltpu.make_async_copy(
                idx_hbm.at[pl.ds(iblk0 + blk * _BLK, _BLK)],
                idx_v.at[buf],
                isem.at[buf],
            )

        def gather(blk_buf, j, b):
            return pltpu.make_async_copy(
                tab_sh.at[idx_v.at[blk_buf, j]], rows_v.at[b], gsem.at[b]
            )

        def write(g, b):
            return pltpu.make_async_copy(
                rows_v.at[b], out_hbm.at[pl.ds(row0 + g * _CH, _CH)], wsem.at[b]
            )

        @pl.when(lax.axis_index("s") == 0)
        def _():
            pltpu.sync_copy(table_hbm, tab_sh)
        plsc.subcore_barrier()

        idx_load(0, 0).start()
        idx_load(1, 1).start()
        idx_load(2, 2).start()
        idx_load(0, 0).wait()
        for j in range(_BLK):
            b = j % _NBUF
            if j >= _NBUF:
                write(j - _NBUF, b).wait()
            gather(0, j, b).start()
            if j >= _SKEW:
                bw = (j - _SKEW) % _NBUF
                gather(0, j - _SKEW, bw).wait()
                write(j - _SKEW, bw).start()

        def block_body(t, carry):
            ib = t % 3
            idx_load(t, ib).wait()
            for j in range(_BLK):
                g = t * _BLK + j
                b = lax.rem(g, _NBUF)
                write(g - _NBUF, b).wait()
                gather(ib, j, b).start()
                bw = lax.rem(g - _SKEW, _NBUF)
                gather(ib, j, bw).wait()
                write(g - _SKEW, bw).start()
                if j == 3:
                    @pl.when(t < n_blocks - 2)
                    def _():
                        idx_load(t + 2, (t + 2) % 3).start()
            return carry

        lax.fori_loop(1, n_blocks, block_body, 0)

        for c in range(n_chunks - _SKEW, n_chunks):
            b = c % _NBUF
            gather(0, 0, b).wait()
            write(c, b).start()
        for c in range(n_chunks - _NBUF, n_chunks):
            write(c, c % _NBUF).wait()

    return k


def kernel(indices, table):
    Bq, S = indices.shape
    V, D = table.shape
    N = Bq * S
    idx2d = indices.reshape(N // _CH, _CH).astype(jnp.int32)
    out = _make_gather(V, D, N)(table.astype(jnp.float32), idx2d)
    return out.reshape(Bq, S, D)
